# Initial kernel scaffold; baseline (speedup 1.0000x reference)
import functools

import jax
import jax.numpy as jnp
from jax import lax
from jax.experimental import pallas as pl
from jax.experimental.pallas import tpu as pltpu
from jax.experimental.pallas import tpu_sc as plsc

N = 10000
E = 320000
D = 128
EPS = 1e-5

NC = 2
NS = 16
NW = NC * NS
C = 128
EPW = -(-E // NW)
NCH = -(-EPW // C)
EPWP = NCH * C
EE = NW * EPWP
N8 = 10240
RPT = N8 // NS
PAD_DST = N


def _sc_agg_body(x_hbm, src_hbm, dst_hbm, agg_out, cnt_out,
                 src_v, dst_v, rows_a, rows_b, ones_v, zrow_v, zcnt_v,
                 sem_a, sem_b, agg_sh, cnt_sh):
    cid = lax.axis_index("c")
    sid = lax.axis_index("s")
    wid = cid * NS + sid

    zero16 = jnp.zeros((16,), jnp.float32)
    one16 = jnp.ones((16,), jnp.float32)
    for r in range(16):
        for c8 in range(8):
            zrow_v[r, pl.ds(c8 * 16, 16)] = zero16
    for r in range(RPT // 16):
        zcnt_v[pl.ds(r * 16, 16)] = zero16
    for r in range(C // 16):
        ones_v[pl.ds(r * 16, 16)] = one16

    base = sid * RPT
    for t in range(RPT // 16):
        pltpu.sync_copy(zrow_v, agg_sh.at[pl.ds(base + t * 16, 16)])
    pltpu.sync_copy(zcnt_v, cnt_sh.at[pl.ds(base, RPT)])

    pltpu.sync_copy(src_hbm.at[wid], src_v)
    pltpu.sync_copy(dst_hbm.at[wid], dst_v)

    plsc.subcore_barrier()

    pltpu.async_copy(x_hbm.at[src_v.at[0]], rows_a, sem_a)

    def pair_body(k, carry):
        c0 = 2 * k
        c1 = c0 + 1
        pltpu.async_copy(x_hbm.at[src_v.at[c1]], rows_b, sem_b)
        pltpu.make_async_copy(x_hbm.at[src_v.at[c0]], rows_a, sem_a).wait()
        pltpu.sync_copy(rows_a, agg_sh.at[dst_v.at[c0]], add=True)
        pltpu.sync_copy(ones_v, cnt_sh.at[dst_v.at[c0]], add=True)
        pltpu.async_copy(x_hbm.at[src_v.at[c0 + 2]], rows_a, sem_a)
        pltpu.make_async_copy(x_hbm.at[src_v.at[c1]], rows_b, sem_b).wait()
        pltpu.sync_copy(rows_b, agg_sh.at[dst_v.at[c1]], add=True)
        pltpu.sync_copy(ones_v, cnt_sh.at[dst_v.at[c1]], add=True)
        return carry

    lax.fori_loop(0, (NCH - 1) // 2, pair_body, 0)
    last = NCH - 1
    pltpu.make_async_copy(x_hbm.at[src_v.at[last]], rows_a, sem_a).wait()
    pltpu.sync_copy(rows_a, agg_sh.at[dst_v.at[last]], add=True)
    pltpu.sync_copy(ones_v, cnt_sh.at[dst_v.at[last]], add=True)

    plsc.subcore_barrier()

    pltpu.sync_copy(agg_sh.at[pl.ds(base, RPT)], agg_out.at[cid, pl.ds(base, RPT)])
    pltpu.sync_copy(cnt_sh.at[pl.ds(base, RPT)], cnt_out.at[cid, pl.ds(base, RPT)])


_sc_agg = pl.kernel(
    _sc_agg_body,
    out_type=[
        jax.ShapeDtypeStruct((NC, N8, D), jnp.float32),
        jax.ShapeDtypeStruct((NC, N8), jnp.float32),
    ],
    mesh=plsc.VectorSubcoreMesh(core_axis_name="c", subcore_axis_name="s"),
    scratch_types=[
        pltpu.VMEM((NCH, C), jnp.int32),
        pltpu.VMEM((NCH, C), jnp.int32),
        pltpu.VMEM((C, D), jnp.float32),
        pltpu.VMEM((C, D), jnp.float32),
        pltpu.VMEM((C,), jnp.float32),
        pltpu.VMEM((16, D), jnp.float32),
        pltpu.VMEM((RPT,), jnp.float32),
        pltpu.SemaphoreType.DMA,
        pltpu.SemaphoreType.DMA,
        pltpu.VMEM_SHARED((N8, D), jnp.float32),
        pltpu.VMEM_SHARED((N8,), jnp.float32),
    ],
)


def _tc_dense_body(aggp_ref, cnt_ref, x_ref, wlt_ref, wrt_ref, b_ref, g_ref,
                   bt_ref, o_ref):
    agg = aggp_ref[0] + aggp_ref[1]
    cnt = jnp.maximum(cnt_ref[0] + cnt_ref[1], 1.0)
    agg = agg / cnt
    h = (jnp.dot(agg, wlt_ref[...], preferred_element_type=jnp.float32)
         + jnp.dot(x_ref[...], wrt_ref[...], preferred_element_type=jnp.float32)
         + b_ref[...])
    mu = jnp.mean(h, axis=-1, keepdims=True)
    var = jnp.mean((h - mu) * (h - mu), axis=-1, keepdims=True)
    h = (h - mu) * lax.rsqrt(var + EPS) * g_ref[...] + bt_ref[...]
    o_ref[...] = jnp.maximum(h, 0.0)


_TC_R = 1000

_tc_dense = pl.pallas_call(
    _tc_dense_body,
    grid=(N // _TC_R,),
    in_specs=[
        pl.BlockSpec((NC, _TC_R, D), lambda i: (0, i, 0)),
        pl.BlockSpec((NC, _TC_R, 1), lambda i: (0, i, 0)),
        pl.BlockSpec((_TC_R, D), lambda i: (i, 0)),
        pl.BlockSpec((D, D), lambda i: (0, 0)),
        pl.BlockSpec((D, D), lambda i: (0, 0)),
        pl.BlockSpec((1, D), lambda i: (0, 0)),
        pl.BlockSpec((1, D), lambda i: (0, 0)),
        pl.BlockSpec((1, D), lambda i: (0, 0)),
    ],
    out_specs=pl.BlockSpec((_TC_R, D), lambda i: (i, 0)),
    out_shape=jax.ShapeDtypeStruct((N, D), jnp.float32),
)


def _sage_layer_opt(x, srcp, dstp, Wl, Wr, b, g, bt):
    aggp, cnt = _sc_agg(x, srcp, dstp)
    return _tc_dense(aggp, cnt.reshape(NC, N8, 1), x, Wl.T, Wr.T,
                     b.reshape(1, D), g.reshape(1, D), bt.reshape(1, D))


def kernel(x, edge_index, Wl0, Wr0, b0, g0, bt0, Wl1, Wr1, b1, g1, bt1):
    pad = EE - E
    srcp = jnp.concatenate(
        [edge_index[0], jnp.zeros((pad,), jnp.int32)]).reshape(NW, NCH, C)
    dstp = jnp.concatenate(
        [edge_index[1], jnp.full((pad,), PAD_DST, jnp.int32)]).reshape(NW, NCH, C)
    h = _sage_layer_opt(x, srcp, dstp, Wl0, Wr0, b0, g0, bt0)
    h = _sage_layer_opt(h, srcp, dstp, Wl1, Wr1, b1, g1, bt1)
    return h


# speedup vs baseline: 6.3032x; 6.3032x over previous
import jax
import jax.numpy as jnp
from jax import lax
from jax.experimental import pallas as pl
from jax.experimental.pallas import tpu as pltpu
from jax.experimental.pallas import tpu_sc as plsc

N = 10000
E = 320000
D = 128
EPS = 1e-5

NC = 2
NS = 16
NW = NC * NS
C = 128
EPT = E // NW
NCH = -(-EPT // C)
EPTP = NCH * C
N8 = 10240
RPT = N8 // NS
PAD_DST = N


def _sc_agg_body(x_hbm, sd_hbm, agg_out, cnt_out,
                 i0, i1, i2, i3, rows_a, rows_b, ones_v, zrow_v, zcnt_v,
                 si0, si1, si2, si3, sra, srb, agg_sh, cnt_sh):
    cid = lax.axis_index("c")
    sid = lax.axis_index("s")
    wid = cid * NS + sid

    ibufs = [i0, i1, i2, i3]
    isems = [si0, si1, si2, si3]
    rbufs = [rows_a, rows_b]
    rsems = [sra, srb]

    zero16 = jnp.zeros((16,), jnp.float32)
    one16 = jnp.ones((16,), jnp.float32)
    for r in range(16):
        for c8 in range(D // 16):
            zrow_v[r, pl.ds(c8 * 16, 16)] = zero16
    for r in range(RPT // 16):
        zcnt_v[pl.ds(r * 16, 16)] = zero16
    for r in range(C // 16):
        ones_v[pl.ds(r * 16, 16)] = one16

    base = sid * RPT
    for t in range(RPT // 16):
        pltpu.sync_copy(zrow_v, agg_sh.at[pl.ds(base + t * 16, 16)])
    pltpu.sync_copy(zcnt_v, cnt_sh.at[pl.ds(base, RPT)])

    plsc.subcore_barrier()

    def idx_load(c, b):
        return pltpu.async_copy(sd_hbm.at[wid, c], ibufs[b], isems[b])

    def idx_wait(b):
        pltpu.make_async_copy(sd_hbm.at[wid, 0], ibufs[b], isems[b]).wait()

    def gather(b, rb):
        return pltpu.async_copy(x_hbm.at[ibufs[b].at[0]], rbufs[rb], rsems[rb])

    def gather_wait(rb):
        pltpu.make_async_copy(x_hbm.at[ibufs[0].at[0]], rbufs[rb],
                              rsems[rb]).wait()

    def scatter(b, rb):
        pltpu.sync_copy(rbufs[rb], agg_sh.at[ibufs[b].at[1]], add=True)
        pltpu.sync_copy(ones_v, cnt_sh.at[ibufs[b].at[1]], add=True)

    for b in range(4):
        idx_load(b, b)
    idx_wait(0)
    gather(0, 0)

    def slot(c, j):
        idx_wait((j + 1) % 4)
        gather((j + 1) % 4, (j + 1) % 2)
        gather_wait(j % 2)
        scatter(j % 4, j % 2)
        idx_load(jnp.minimum(c + 4, NCH - 1), j % 4)

    def quad_body(k, carry):
        for j in range(4):
            slot(4 * k + j, j)
        return carry

    nquad = (NCH - 2) // 4
    lax.fori_loop(0, nquad, quad_body, 0)
    for c in range(4 * nquad, NCH - 2):
        slot(jnp.int32(c), c % 4)
    c = NCH - 2
    idx_wait((c + 1) % 4)
    gather((c + 1) % 4, (c + 1) % 2)
    gather_wait(c % 2)
    scatter(c % 4, c % 2)
    c = NCH - 1
    gather_wait(c % 2)
    scatter(c % 4, c % 2)
    idx_wait((NCH - 4) % 4)
    idx_wait((NCH - 3) % 4)

    plsc.subcore_barrier()

    pltpu.sync_copy(agg_sh.at[pl.ds(base, RPT)], agg_out.at[cid, pl.ds(base, RPT)])
    pltpu.sync_copy(cnt_sh.at[pl.ds(base, RPT)], cnt_out.at[cid, pl.ds(base, RPT)])


_sc_agg = pl.kernel(
    _sc_agg_body,
    out_type=[
        jax.ShapeDtypeStruct((NC, N8, D), jnp.float32),
        jax.ShapeDtypeStruct((NC, N8), jnp.float32),
    ],
    mesh=plsc.VectorSubcoreMesh(core_axis_name="c", subcore_axis_name="s"),
    scratch_types=[
        pltpu.VMEM((2, C), jnp.int32),
        pltpu.VMEM((2, C), jnp.int32),
        pltpu.VMEM((2, C), jnp.int32),
        pltpu.VMEM((2, C), jnp.int32),
        pltpu.VMEM((C, D), jnp.float32),
        pltpu.VMEM((C, D), jnp.float32),
        pltpu.VMEM((C,), jnp.float32),
        pltpu.VMEM((16, D), jnp.float32),
        pltpu.VMEM((RPT,), jnp.float32),
        pltpu.SemaphoreType.DMA,
        pltpu.SemaphoreType.DMA,
        pltpu.SemaphoreType.DMA,
        pltpu.SemaphoreType.DMA,
        pltpu.SemaphoreType.DMA,
        pltpu.SemaphoreType.DMA,
        pltpu.VMEM_SHARED((N8, D), jnp.float32),
        pltpu.VMEM_SHARED((N8,), jnp.float32),
    ],
)


def _tc_dense_body(aggp_ref, cnt_ref, x_ref, wlt_ref, wrt_ref, b_ref, g_ref,
                   bt_ref, o_ref):
    agg = aggp_ref[0] + aggp_ref[1]
    cnt = jnp.maximum(cnt_ref[0] + cnt_ref[1], 1.0)
    agg = agg / cnt
    h = (jnp.dot(agg, wlt_ref[...], preferred_element_type=jnp.float32)
         + jnp.dot(x_ref[...], wrt_ref[...], preferred_element_type=jnp.float32)
         + b_ref[...])
    mu = jnp.mean(h, axis=-1, keepdims=True)
    var = jnp.mean((h - mu) * (h - mu), axis=-1, keepdims=True)
    h = (h - mu) * lax.rsqrt(var + EPS) * g_ref[...] + bt_ref[...]
    o_ref[...] = jnp.maximum(h, 0.0)


_TC_R = 1000

_tc_dense = pl.pallas_call(
    _tc_dense_body,
    grid=(N // _TC_R,),
    in_specs=[
        pl.BlockSpec((NC, _TC_R, D), lambda i: (0, i, 0)),
        pl.BlockSpec((NC, _TC_R, 1), lambda i: (0, i, 0)),
        pl.BlockSpec((_TC_R, D), lambda i: (i, 0)),
        pl.BlockSpec((D, D), lambda i: (0, 0)),
        pl.BlockSpec((D, D), lambda i: (0, 0)),
        pl.BlockSpec((1, D), lambda i: (0, 0)),
        pl.BlockSpec((1, D), lambda i: (0, 0)),
        pl.BlockSpec((1, D), lambda i: (0, 0)),
    ],
    out_specs=pl.BlockSpec((_TC_R, D), lambda i: (i, 0)),
    out_shape=jax.ShapeDtypeStruct((N, D), jnp.float32),
)


def _sage_layer_opt(x, sd, Wl, Wr, b, g, bt):
    aggp, cnt = _sc_agg(x, sd)
    return _tc_dense(aggp, cnt.reshape(NC, N8, 1), x, Wl.T, Wr.T,
                     b.reshape(1, D), g.reshape(1, D), bt.reshape(1, D))


def kernel(x, edge_index, Wl0, Wr0, b0, g0, bt0, Wl1, Wr1, b1, g1, bt1):
    pad = EPTP - EPT
    srcp = jnp.concatenate(
        [edge_index[0].reshape(NW, EPT),
         jnp.zeros((NW, pad), jnp.int32)], axis=1).reshape(NW, NCH, C)
    dstp = jnp.concatenate(
        [edge_index[1].reshape(NW, EPT),
         jnp.full((NW, pad), PAD_DST, jnp.int32)], axis=1).reshape(NW, NCH, C)
    sd = jnp.stack([srcp, dstp], axis=2)
    h = _sage_layer_opt(x, sd, Wl0, Wr0, b0, g0, bt0)
    h = _sage_layer_opt(h, sd, Wl1, Wr1, b1, g1, bt1)
    return h


# speedup vs baseline: 7.6968x; 1.2211x over previous
import functools

import jax
import jax.numpy as jnp
from jax import lax
from jax.experimental import pallas as pl
from jax.experimental.pallas import tpu as pltpu
from jax.experimental.pallas import tpu_sc as plsc

N = 10000
E = 320000
D = 128
EPS = 1e-5

NC = 2
NS = 16
NW = NC * NS
C = 96
NI = 6
NR = 3
EPT = E // NW
NCH = -(-EPT // C)
EPTP = NCH * C
N8 = 10240
RPT = N8 // NS
PAD_DST = N


def _sc_agg_body(with_cnt, x_hbm, sd_hbm, *refs):
    if with_cnt:
        (agg_out, cnt_out, i0, i1, i2, i3, i4, i5, ra, rb, rc, ones_v,
         zrow_v, zcnt_v, si0, si1, si2, si3, si4, si5, sr0, sr1, sr2,
         ss0, ss1, ss2, sc0, sc1, sc2, agg_sh, cnt_sh) = refs
        csems = [sc0, sc1, sc2]
    else:
        (agg_out, i0, i1, i2, i3, i4, i5, ra, rb, rc, ones_v,
         zrow_v, si0, si1, si2, si3, si4, si5, sr0, sr1, sr2,
         ss0, ss1, ss2, agg_sh) = refs
        cnt_out = cnt_sh = zcnt_v = None
        csems = None
    ibufs = [i0, i1, i2, i3, i4, i5]
    isems = [si0, si1, si2, si3, si4, si5]
    rbufs = [ra, rb, rc]
    rsems = [sr0, sr1, sr2]
    ssems = [ss0, ss1, ss2]

    cid = lax.axis_index("c")
    sid = lax.axis_index("s")
    wid = cid * NS + sid

    zero16 = jnp.zeros((16,), jnp.float32)
    one16 = jnp.ones((16,), jnp.float32)
    for r in range(16):
        for c8 in range(D // 16):
            zrow_v[r, pl.ds(c8 * 16, 16)] = zero16
    for r in range(C // 16):
        ones_v[pl.ds(r * 16, 16)] = one16

    base = sid * RPT
    for t in range(RPT // 16):
        pltpu.sync_copy(zrow_v, agg_sh.at[pl.ds(base + t * 16, 16)])
    if with_cnt:
        for r in range(RPT // 16):
            zcnt_v[pl.ds(r * 16, 16)] = zero16
        pltpu.sync_copy(zcnt_v, cnt_sh.at[pl.ds(base, RPT)])

    plsc.subcore_barrier()

    def idx_load(c, b):
        pltpu.async_copy(sd_hbm.at[wid, c], ibufs[b], isems[b])

    def idx_wait(b):
        pltpu.make_async_copy(sd_hbm.at[wid, 0], ibufs[b], isems[b]).wait()

    def scatter_wait(b):
        pltpu.make_async_copy(rbufs[b], agg_sh.at[ibufs[0].at[1]],
                              ssems[b]).wait()

    def cnt_wait(b):
        pltpu.make_async_copy(ones_v, cnt_sh.at[ibufs[0].at[1]],
                              csems[b]).wait()

    def slot(c, cm6, do_ws, do_il, do_g, do_cw):
        cm3 = cm6 % 3
        if do_ws:
            scatter_wait((cm3 + 1) % 3)
        if do_il:
            idx_load(c + 4, (cm6 + 4) % 6)
        if do_g:
            idx_wait((cm6 + 1) % 6)
            pltpu.async_copy(x_hbm.at[ibufs[(cm6 + 1) % 6].at[0]],
                             rbufs[(cm3 + 1) % 3], rsems[(cm3 + 1) % 3])
        pltpu.make_async_copy(x_hbm.at[ibufs[0].at[0]], rbufs[cm3],
                              rsems[cm3]).wait()
        pltpu.async_copy(rbufs[cm3], agg_sh.at[ibufs[cm6].at[1]],
                         ssems[cm3], add=True)
        if with_cnt:
            if do_cw:
                cnt_wait(cm3)
            pltpu.async_copy(ones_v, cnt_sh.at[ibufs[cm6].at[1]],
                             csems[cm3], add=True)

    for c in range(4):
        idx_load(c, c)
    idx_wait(0)
    pltpu.async_copy(x_hbm.at[ibufs[0].at[0]], rbufs[0], rsems[0])

    slot(0, 0, False, True, True, False)
    slot(1, 1, False, True, True, False)
    slot(2, 2, True, True, True, False)
    slot(3, 3, True, True, True, True)
    slot(4, 4, True, True, True, True)
    slot(5, 5, True, True, True, True)

    def six_body(k, carry):
        cbase = 6 * k
        for o in range(6):
            slot(cbase + o, o, True, True, True, True)
        return carry

    lax.fori_loop(1, (NCH - 3) // 6, six_body, 0)

    slot(NCH - 3, (NCH - 3) % 6, True, False, True, True)
    slot(NCH - 2, (NCH - 2) % 6, True, False, True, True)
    slot(NCH - 1, (NCH - 1) % 6, True, False, False, True)

    scatter_wait((NCH - 2) % 3)
    scatter_wait((NCH - 1) % 3)
    if with_cnt:
        for cc in range(NCH - 3, NCH):
            cnt_wait(cc % 3)

    plsc.subcore_barrier()

    pltpu.sync_copy(agg_sh.at[pl.ds(base, RPT)], agg_out.at[cid, pl.ds(base, RPT)])
    if with_cnt:
        pltpu.sync_copy(cnt_sh.at[pl.ds(base, RPT)], cnt_out.at[cid, pl.ds(base, RPT)])


def _make_sc_agg(with_cnt):
    out_type = [jax.ShapeDtypeStruct((NC, N8, D), jnp.float32)]
    if with_cnt:
        out_type.append(jax.ShapeDtypeStruct((NC, N8), jnp.float32))
    scratch = [pltpu.VMEM((2, C), jnp.int32) for _ in range(NI)]
    scratch += [pltpu.VMEM((C, D), jnp.float32) for _ in range(NR)]
    scratch += [pltpu.VMEM((C,), jnp.float32),
                pltpu.VMEM((16, D), jnp.float32)]
    if with_cnt:
        scratch.append(pltpu.VMEM((RPT,), jnp.float32))
    nsem = NI + NR + NR + (NR if with_cnt else 0)
    scratch += [pltpu.SemaphoreType.DMA for _ in range(nsem)]
    scratch.append(pltpu.VMEM_SHARED((N8, D), jnp.float32))
    if with_cnt:
        scratch.append(pltpu.VMEM_SHARED((N8,), jnp.float32))
    return pl.kernel(
        functools.partial(_sc_agg_body, with_cnt),
        out_type=out_type,
        mesh=plsc.VectorSubcoreMesh(core_axis_name="c", subcore_axis_name="s"),
        scratch_types=scratch,
    )


_sc_agg_cnt = _make_sc_agg(True)
_sc_agg_nc = _make_sc_agg(False)


def _tc_dense_body(aggp_ref, cnt_ref, x_ref, wlt_ref, wrt_ref, b_ref, g_ref,
                   bt_ref, o_ref):
    agg = aggp_ref[0] + aggp_ref[1]
    cnt = jnp.maximum(cnt_ref[0] + cnt_ref[1], 1.0)
    agg = agg / cnt
    h = (jnp.dot(agg, wlt_ref[...], preferred_element_type=jnp.float32)
         + jnp.dot(x_ref[...], wrt_ref[...], preferred_element_type=jnp.float32)
         + b_ref[...])
    mu = jnp.mean(h, axis=-1, keepdims=True)
    var = jnp.mean((h - mu) * (h - mu), axis=-1, keepdims=True)
    h = (h - mu) * lax.rsqrt(var + EPS) * g_ref[...] + bt_ref[...]
    o_ref[...] = jnp.maximum(h, 0.0)


_TC_R = 1000

_tc_dense = pl.pallas_call(
    _tc_dense_body,
    grid=(N // _TC_R,),
    in_specs=[
        pl.BlockSpec((NC, _TC_R, D), lambda i: (0, i, 0)),
        pl.BlockSpec((NC, _TC_R, 1), lambda i: (0, i, 0)),
        pl.BlockSpec((_TC_R, D), lambda i: (i, 0)),
        pl.BlockSpec((D, D), lambda i: (0, 0)),
        pl.BlockSpec((D, D), lambda i: (0, 0)),
        pl.BlockSpec((1, D), lambda i: (0, 0)),
        pl.BlockSpec((1, D), lambda i: (0, 0)),
        pl.BlockSpec((1, D), lambda i: (0, 0)),
    ],
    out_specs=pl.BlockSpec((_TC_R, D), lambda i: (i, 0)),
    out_shape=jax.ShapeDtypeStruct((N, D), jnp.float32),
)


def kernel(x, edge_index, Wl0, Wr0, b0, g0, bt0, Wl1, Wr1, b1, g1, bt1):
    pad = EPTP - EPT
    srcp = jnp.concatenate(
        [edge_index[0].reshape(NW, EPT),
         jnp.zeros((NW, pad), jnp.int32)], axis=1).reshape(NW, NCH, C)
    dstp = jnp.concatenate(
        [edge_index[1].reshape(NW, EPT),
         jnp.full((NW, pad), PAD_DST, jnp.int32)], axis=1).reshape(NW, NCH, C)
    sd = jnp.stack([srcp, dstp], axis=2)
    cnt3 = None
    aggp, cnt = _sc_agg_cnt(x, sd)
    cnt3 = cnt.reshape(NC, N8, 1)
    h = _tc_dense(aggp, cnt3, x, Wl0.T, Wr0.T,
                  b0.reshape(1, D), g0.reshape(1, D), bt0.reshape(1, D))
    (aggp2,) = _sc_agg_nc(h, sd)
    h = _tc_dense(aggp2, cnt3, h, Wl1.T, Wr1.T,
                  b1.reshape(1, D), g1.reshape(1, D), bt1.reshape(1, D))
    return h
